# Initial kernel scaffold; baseline (speedup 1.0000x reference)
import jax
import jax.numpy as jnp
from jax.experimental import pallas as pl


def kernel(actions, hidden, edge_index, W_ih, W_hh, b_ih, b_hh, ln_g, ln_b, gcn_W, gcn_b, zIA_mu_W, zIA_mu_b, zIA_std_W, zIA_std_b, zIG_mu_W, zIG_mu_b, zIG_std_W, zIG_std_b):
    raise NotImplementedError("write your pallas kernel here")


# speedup vs baseline: 7.1888x; 7.1888x over previous
import functools

import jax
import jax.numpy as jnp
from jax import lax
from jax.experimental import pallas as pl
from jax.experimental.pallas import tpu as pltpu
from jax.experimental.pallas import tpu_sc as plsc

N = 10000
E = 320000
T = 8
A = 16
H = 128
I = 64

NTILES = 32
LANES = 16
CHUNK = 64
CH = -(-E // (NTILES * CHUNK))
EPT = CH * CHUNK
EP = NTILES * EPT
PAD = EP - E
RPT = 626
NP = 16 * RPT
NB = 1000
GRID = N // NB

_mesh = plsc.VectorSubcoreMesh(core_axis_name="c", subcore_axis_name="s")
_sc_params = pltpu.CompilerParams(needs_layout_passes=False,
                                  use_tc_tiling_on_sc=False)


@functools.partial(
    pl.kernel,
    out_type=(jax.ShapeDtypeStruct((NTILES, N), jnp.float32),
              jax.ShapeDtypeStruct((NTILES, N), jnp.float32)),
    mesh=_mesh,
    scratch_types=[
        pltpu.VMEM((EPT,), jnp.int32),
        pltpu.VMEM((EPT,), jnp.int32),
        pltpu.VMEM((NP,), jnp.float32),
        pltpu.VMEM((NP,), jnp.float32),
    ],
    compiler_params=_sc_params,
)
def _sc_degrees(src_hbm, dst_hbm, dego_hbm, degi_hbm, src_v, dst_v, ho_v, hi_v):
    c = lax.axis_index("c")
    s = lax.axis_index("s")
    wid = c * 16 + s
    pltpu.sync_copy(src_hbm.at[wid], src_v)
    pltpu.sync_copy(dst_hbm.at[wid], dst_v)
    zeros = jnp.zeros((LANES,), jnp.float32)
    ones = jnp.ones((LANES,), jnp.float32)

    def zero_body(v, _):
        ho_v[pl.ds(v * LANES, LANES)] = zeros
        hi_v[pl.ds(v * LANES, LANES)] = zeros
        return _

    lax.fori_loop(0, NP // LANES, zero_body, 0)

    def hist_body(v, _):
        si = src_v[pl.ds(v * LANES, LANES)]
        di = dst_v[pl.ds(v * LANES, LANES)]
        plsc.addupdate_scatter(ho_v, [si], ones)
        plsc.addupdate_scatter(hi_v, [di], ones)
        return _

    lax.fori_loop(0, EPT // LANES, hist_body, 0)
    pltpu.sync_copy(ho_v.at[pl.ds(0, N)], dego_hbm.at[wid])
    pltpu.sync_copy(hi_v.at[pl.ds(0, N)], degi_hbm.at[wid])


def _tc_gru_body(a_ref, h0_ref, wih_ref, whh_ref, bih_ref, bhh_ref,
                 lng_ref, lnb_ref, degp_ref, hfin_ref, xsc_ref):
    blk = pl.program_id(0)
    wih = wih_ref[...]
    whh = whh_ref[...]
    bih = bih_ref[...]
    bhh = bhh_ref[...]
    h = h0_ref[0]
    for t in range(T):
        x_t = a_ref[t]
        gi = jnp.dot(x_t, wih, preferred_element_type=jnp.float32) + bih
        gh = jnp.dot(h, whh, preferred_element_type=jnp.float32) + bhh
        i_r = gi[:, :H]
        i_z = gi[:, H:2 * H]
        i_n = gi[:, 2 * H:]
        h_r = gh[:, :H]
        h_z = gh[:, H:2 * H]
        h_n = gh[:, 2 * H:]
        r = jax.nn.sigmoid(i_r + h_r)
        z = jax.nn.sigmoid(i_z + h_z)
        n = jnp.tanh(i_n + r * h_n)
        h = (1.0 - z) * n + z * h
    hfin_ref[...] = h

    mu = jnp.mean(h, axis=1, keepdims=True)
    d = h - mu
    var = jnp.mean(d * d, axis=1, keepdims=True)
    x = d * lax.rsqrt(var + 1e-5) * lng_ref[...] + lnb_ref[...]

    deg = jnp.sum(degp_ref[...], axis=1).reshape(NB, 1)
    row = jax.lax.broadcasted_iota(jnp.int32, (NB, 1), 0) + blk * NB
    deg = jnp.where(row == 0, deg - float(PAD), deg)
    norm_src = lax.rsqrt(jnp.maximum(deg, 1.0))
    xsc_ref[...] = x * norm_src


def _tc_gru(a_t, hidden, wihT, whhT, bih, bhh, lng, lnb, deg_out_parts):
    return pl.pallas_call(
        _tc_gru_body,
        grid=(GRID,),
        in_specs=[
            pl.BlockSpec((T, NB, A), lambda i: (0, i, 0)),
            pl.BlockSpec((1, NB, H), lambda i: (0, i, 0)),
            pl.BlockSpec((A, 3 * H), lambda i: (0, 0)),
            pl.BlockSpec((H, 3 * H), lambda i: (0, 0)),
            pl.BlockSpec((1, 3 * H), lambda i: (0, 0)),
            pl.BlockSpec((1, 3 * H), lambda i: (0, 0)),
            pl.BlockSpec((1, H), lambda i: (0, 0)),
            pl.BlockSpec((1, H), lambda i: (0, 0)),
            pl.BlockSpec((NB, NTILES), lambda i: (i, 0)),
        ],
        out_specs=[
            pl.BlockSpec((NB, H), lambda i: (i, 0)),
            pl.BlockSpec((NB, H), lambda i: (i, 0)),
        ],
        out_shape=[
            jax.ShapeDtypeStruct((N, H), jnp.float32),
            jax.ShapeDtypeStruct((N, H), jnp.float32),
        ],
    )(a_t, hidden, wihT, whhT, bih, bhh, lng, lnb, deg_out_parts)


@functools.partial(
    pl.kernel,
    out_type=jax.ShapeDtypeStruct((2, NP, H), jnp.float32),
    mesh=_mesh,
    scratch_types=[
        pltpu.VMEM((CH, CHUNK), jnp.int32),
        pltpu.VMEM((CH, CHUNK), jnp.int32),
        pltpu.VMEM((CHUNK, H), jnp.float32),
        pltpu.VMEM((CHUNK, H), jnp.float32),
        pltpu.VMEM_SHARED((NP, H), jnp.float32),
        pltpu.SemaphoreType.DMA,
        pltpu.SemaphoreType.DMA,
    ],
    compiler_params=_sc_params,
)
def _sc_gather_scatter(x_hbm, src_hbm, dst_hbm, zeros_hbm, out_hbm,
                       src_v, dst_v, buf0, buf1, agg_sh, sem0, sem1):
    c = lax.axis_index("c")
    s = lax.axis_index("s")
    wid = c * 16 + s
    pltpu.sync_copy(zeros_hbm, agg_sh.at[pl.ds(s * RPT, RPT)])
    pltpu.sync_copy(src_hbm.at[wid], src_v)
    pltpu.sync_copy(dst_hbm.at[wid], dst_v)
    plsc.subcore_barrier()

    cp0 = pltpu.async_copy(x_hbm.at[src_v.at[0]], buf0, sem0)

    def body(i, _):
        ci = 2 * i
        cp1 = pltpu.async_copy(x_hbm.at[src_v.at[ci + 1]], buf1, sem1)
        pltpu.make_async_copy(x_hbm.at[src_v.at[ci]], buf0, sem0).wait()
        pltpu.sync_copy(buf0, agg_sh.at[dst_v.at[ci]], add=True)
        nxt = pltpu.async_copy(x_hbm.at[src_v.at[ci + 2]], buf0, sem0)
        cp1.wait()
        pltpu.sync_copy(buf1, agg_sh.at[dst_v.at[ci + 1]], add=True)
        return _

    lax.fori_loop(0, (CH - 1) // 2, body, 0)
    pltpu.make_async_copy(x_hbm.at[src_v.at[CH - 1]], buf0, sem0).wait()
    pltpu.sync_copy(buf0, agg_sh.at[dst_v.at[CH - 1]], add=True)

    plsc.subcore_barrier()
    pltpu.sync_copy(agg_sh.at[pl.ds(s * RPT, RPT)],
                    out_hbm.at[c, pl.ds(s * RPT, RPT)])


def _tc_heads_body(agg2_ref, degp_ref, gw_ref, gb_ref,
                   wam_ref, bam_ref, was_ref, bas_ref,
                   wgm_ref, bgm_ref, wgs_ref, bgs_ref,
                   am_ref, as_ref, gm_ref, gs_ref):
    agg = agg2_ref[0] + agg2_ref[1]
    deg = jnp.sum(degp_ref[...], axis=1).reshape(NB, 1)
    norm_dst = lax.rsqrt(jnp.maximum(deg, 1.0))
    hg = jnp.dot(agg * norm_dst, gw_ref[...],
                 preferred_element_type=jnp.float32) + gb_ref[...]
    am_ref[...] = jnp.dot(hg, wam_ref[...],
                          preferred_element_type=jnp.float32) + bam_ref[...]
    as_ref[...] = jax.nn.softplus(
        jnp.dot(hg, was_ref[...], preferred_element_type=jnp.float32)
        + bas_ref[...])
    gm_ref[...] = jnp.dot(hg, wgm_ref[...],
                          preferred_element_type=jnp.float32) + bgm_ref[...]
    gs_ref[...] = jax.nn.softplus(
        jnp.dot(hg, wgs_ref[...], preferred_element_type=jnp.float32)
        + bgs_ref[...])


def _tc_heads(agg2, deg_in_parts, gcn_W, gcn_b,
              wamT, bam, wasT, bas, wgmT, bgm, wgsT, bgs):
    return pl.pallas_call(
        _tc_heads_body,
        grid=(GRID,),
        in_specs=[
            pl.BlockSpec((2, NB, H), lambda i: (0, i, 0)),
            pl.BlockSpec((NB, NTILES), lambda i: (i, 0)),
            pl.BlockSpec((H, H), lambda i: (0, 0)),
            pl.BlockSpec((1, H), lambda i: (0, 0)),
            pl.BlockSpec((H, I), lambda i: (0, 0)),
            pl.BlockSpec((1, I), lambda i: (0, 0)),
            pl.BlockSpec((H, I), lambda i: (0, 0)),
            pl.BlockSpec((1, I), lambda i: (0, 0)),
            pl.BlockSpec((H, I), lambda i: (0, 0)),
            pl.BlockSpec((1, I), lambda i: (0, 0)),
            pl.BlockSpec((H, I), lambda i: (0, 0)),
            pl.BlockSpec((1, I), lambda i: (0, 0)),
        ],
        out_specs=[pl.BlockSpec((NB, I), lambda i: (i, 0))] * 4,
        out_shape=[jax.ShapeDtypeStruct((N, I), jnp.float32)] * 4,
    )(agg2, deg_in_parts, gcn_W, gcn_b, wamT, bam, wasT, bas, wgmT, bgm, wgsT, bgs)


def kernel(actions, hidden, edge_index, W_ih, W_hh, b_ih, b_hh, ln_g, ln_b,
           gcn_W, gcn_b, zIA_mu_W, zIA_mu_b, zIA_std_W, zIA_std_b,
           zIG_mu_W, zIG_mu_b, zIG_std_W, zIG_std_b):
    f32 = jnp.float32
    src = edge_index[0]
    dst = edge_index[1]
    src_p = jnp.concatenate([src, jnp.zeros((PAD,), jnp.int32)])
    dst_p = jnp.concatenate([dst, jnp.full((PAD,), N, jnp.int32)])
    src_t = src_p.reshape(NTILES, EPT)
    dst_t = dst_p.reshape(NTILES, EPT)

    dego_parts, degi_parts = _sc_degrees(src_t, dst_t)
    dego_parts = dego_parts.T
    degi_parts = degi_parts.T

    a_t = jnp.transpose(actions, (1, 0, 2))
    h_final, x_scaled = _tc_gru(
        a_t, hidden, W_ih.T, W_hh.T,
        b_ih.reshape(1, 3 * H), b_hh.reshape(1, 3 * H),
        ln_g.reshape(1, H), ln_b.reshape(1, H), dego_parts)

    zeros = jnp.zeros((RPT, H), f32)
    agg2 = _sc_gather_scatter(x_scaled,
                              src_t.reshape(NTILES, CH, CHUNK),
                              dst_t.reshape(NTILES, CH, CHUNK),
                              zeros)

    zIA_mu, zIA_std, zIG_mu, zIG_std = _tc_heads(
        agg2, degi_parts, gcn_W, gcn_b.reshape(1, H),
        zIA_mu_W.T, zIA_mu_b.reshape(1, I),
        zIA_std_W.T, zIA_std_b.reshape(1, I),
        zIG_mu_W.T, zIG_mu_b.reshape(1, I),
        zIG_std_W.T, zIG_std_b.reshape(1, I))

    return zIG_mu, zIG_std, zIA_mu, zIA_std, h_final[None]
